# Initial kernel scaffold; baseline (speedup 1.0000x reference)
import jax
import jax.numpy as jnp
from jax.experimental import pallas as pl


def kernel(z, edge_index):
    raise NotImplementedError("write your pallas kernel here")


# speedup vs baseline: 1.3311x; 1.3311x over previous
import functools

import jax
import jax.numpy as jnp
from jax import lax
from jax.experimental import pallas as pl
from jax.experimental.pallas import tpu as pltpu
from jax.experimental.pallas import tpu_sc as plsc

_LANES = 16
_NW = 32
_NC = 2
_CHUNK = 80


@functools.partial(jax.jit, static_argnames=("n_edges", "n_chunks"))
def _decode(z, src2d, dst2d, *, n_edges, n_chunks):
    mesh = plsc.VectorSubcoreMesh(core_axis_name="c", subcore_axis_name="s")

    @functools.partial(
        pl.kernel,
        mesh=mesh,
        compiler_params=pltpu.CompilerParams(needs_layout_passes=False),
        out_type=jax.ShapeDtypeStruct((_NW, n_chunks, _CHUNK), jnp.float32),
        scratch_types=dict(
            src_idx=pltpu.VMEM((n_chunks, _CHUNK), jnp.int32),
            dst_idx=pltpu.VMEM((n_chunks, _CHUNK), jnp.int32),
            rs_a=pltpu.VMEM((_CHUNK, 128), jnp.float32),
            rd_a=pltpu.VMEM((_CHUNK, 128), jnp.float32),
            rs_b=pltpu.VMEM((_CHUNK, 128), jnp.float32),
            rd_b=pltpu.VMEM((_CHUNK, 128), jnp.float32),
            out_v=pltpu.VMEM((n_chunks, _CHUNK), jnp.float32),
            sem_a=pltpu.SemaphoreType.DMA,
            sem_b=pltpu.SemaphoreType.DMA,
        ),
    )
    def body(z_hbm, src_hbm, dst_hbm, out_hbm,
             src_idx, dst_idx, rs_a, rd_a, rs_b, rd_b, out_v, sem_a, sem_b):
        wid = lax.axis_index("s") * _NC + lax.axis_index("c")

        pltpu.sync_copy(src_hbm.at[wid], src_idx)
        pltpu.sync_copy(dst_hbm.at[wid], dst_idx)

        def issue(l, rs, rd, sem):
            pltpu.async_copy(z_hbm.at[src_idx.at[l]], rs, sem)
            pltpu.async_copy(z_hbm.at[dst_idx.at[l]], rd, sem)

        def drain(l, rs, rd, sem):
            pltpu.make_async_copy(z_hbm.at[src_idx.at[l]], rs, sem).wait()
            pltpu.make_async_copy(z_hbm.at[dst_idx.at[l]], rd, sem).wait()

        def compute(l, rs, rd):
            out_row = out_v.at[l]
            for g in range(_CHUNK // _LANES):
                row = g * _LANES + lax.iota(jnp.int32, _LANES)
                acc0 = jnp.zeros((_LANES,), jnp.float32)
                col0 = jnp.zeros((_LANES,), jnp.int32)

                def dstep(_, carry):
                    acc, col = carry
                    for _k in range(8):
                        s = plsc.load_gather(rs, [row, col])
                        t = plsc.load_gather(rd, [row, col])
                        acc = acc + s * t
                        col = col + 1
                    return acc, col

                acc, _ = lax.fori_loop(0, 128 // 8, dstep, (acc0, col0))
                out_row[pl.ds(g * _LANES, _LANES)] = 1.0 / (1.0 + jnp.exp(-acc))

        issue(0, rs_a, rd_a, sem_a)
        issue(1, rs_b, rd_b, sem_b)
        n_pairs = (n_chunks - 1) // 2

        def pair(j, carry):
            drain(2 * j, rs_a, rd_a, sem_a)
            compute(2 * j, rs_a, rd_a)
            issue(2 * j + 2, rs_a, rd_a, sem_a)
            drain(2 * j + 1, rs_b, rd_b, sem_b)
            compute(2 * j + 1, rs_b, rd_b)

            @pl.when(j < n_pairs - 1)
            def _():
                issue(2 * j + 3, rs_b, rd_b, sem_b)

            return carry

        lax.fori_loop(0, n_pairs, pair, 0)
        drain(n_chunks - 1, rs_a, rd_a, sem_a)
        compute(n_chunks - 1, rs_a, rd_a)

        pltpu.sync_copy(out_v, out_hbm.at[wid])

    return body(z, src2d, dst2d)


def kernel(z, edge_index):
    n_edges = edge_index.shape[1]
    n_chunks = n_edges // (_NW * _CHUNK)
    ei = edge_index.astype(jnp.int32)
    src3d = ei[0].reshape(_NW, n_chunks, _CHUNK)
    dst3d = ei[1].reshape(_NW, n_chunks, _CHUNK)
    out3d = _decode(z, src3d, dst3d, n_edges=n_edges, n_chunks=n_chunks)
    return out3d.reshape(n_edges)


# speedup vs baseline: 1.3388x; 1.0058x over previous
import functools

import jax
import jax.numpy as jnp
from jax import lax
from jax.experimental import pallas as pl
from jax.experimental.pallas import tpu as pltpu
from jax.experimental.pallas import tpu_sc as plsc

_LANES = 16
_NW = 32
_NC = 2
_CHUNK = 80


@functools.partial(jax.jit, static_argnames=("n_edges", "n_chunks"))
def _decode(z, src2d, dst2d, *, n_edges, n_chunks):
    mesh = plsc.VectorSubcoreMesh(core_axis_name="c", subcore_axis_name="s")

    @functools.partial(
        pl.kernel,
        mesh=mesh,
        compiler_params=pltpu.CompilerParams(needs_layout_passes=False),
        out_type=jax.ShapeDtypeStruct((_NW, n_chunks, _CHUNK), jnp.float32),
        scratch_types=dict(
            src_idx=pltpu.VMEM((n_chunks, _CHUNK), jnp.int32),
            dst_idx=pltpu.VMEM((n_chunks, _CHUNK), jnp.int32),
            rs_a=pltpu.VMEM((_CHUNK, 128), jnp.float32),
            rd_a=pltpu.VMEM((_CHUNK, 128), jnp.float32),
            rs_b=pltpu.VMEM((_CHUNK, 128), jnp.float32),
            rd_b=pltpu.VMEM((_CHUNK, 128), jnp.float32),
            out_v=pltpu.VMEM((n_chunks, _CHUNK), jnp.float32),
            sem_a=pltpu.SemaphoreType.DMA,
            sem_b=pltpu.SemaphoreType.DMA,
        ),
    )
    def body(z_hbm, src_hbm, dst_hbm, out_hbm,
             src_idx, dst_idx, rs_a, rd_a, rs_b, rd_b, out_v, sem_a, sem_b):
        wid = lax.axis_index("s") * _NC + lax.axis_index("c")

        pltpu.sync_copy(src_hbm.at[wid], src_idx)
        pltpu.sync_copy(dst_hbm.at[wid], dst_idx)

        def issue(l, rs, rd, sem):
            pltpu.async_copy(z_hbm.at[src_idx.at[l]], rs, sem)
            pltpu.async_copy(z_hbm.at[dst_idx.at[l]], rd, sem)

        def drain(l, rs, rd, sem):
            pltpu.make_async_copy(z_hbm.at[src_idx.at[l]], rs, sem).wait()
            pltpu.make_async_copy(z_hbm.at[dst_idx.at[l]], rd, sem).wait()

        def compute(l, rs, rd):
            out_row = out_v.at[l]
            zero = jnp.zeros((_LANES,), jnp.float32)
            for g in range(_CHUNK // _LANES):
                row = g * _LANES + lax.iota(jnp.int32, _LANES)

                def dstep(i, accs):
                    a0, a1, a2, a3 = accs
                    cb = jnp.full((_LANES,), i * 16, jnp.int32)
                    prods = []
                    for k in range(16):
                        col = cb + k
                        s = plsc.load_gather(rs, [row, col])
                        t = plsc.load_gather(rd, [row, col])
                        prods.append(s * t)
                    for k in range(0, 16, 4):
                        a0 = a0 + prods[k]
                        a1 = a1 + prods[k + 1]
                        a2 = a2 + prods[k + 2]
                        a3 = a3 + prods[k + 3]
                    return a0, a1, a2, a3

                a0, a1, a2, a3 = lax.fori_loop(0, 128 // 16, dstep,
                                               (zero, zero, zero, zero))
                acc = (a0 + a1) + (a2 + a3)
                out_row[pl.ds(g * _LANES, _LANES)] = 1.0 / (1.0 + jnp.exp(-acc))

        issue(0, rs_a, rd_a, sem_a)
        issue(1, rs_b, rd_b, sem_b)
        n_pairs = (n_chunks - 1) // 2

        def pair(j, carry):
            drain(2 * j, rs_a, rd_a, sem_a)
            compute(2 * j, rs_a, rd_a)
            issue(2 * j + 2, rs_a, rd_a, sem_a)
            drain(2 * j + 1, rs_b, rd_b, sem_b)
            compute(2 * j + 1, rs_b, rd_b)

            @pl.when(j < n_pairs - 1)
            def _():
                issue(2 * j + 3, rs_b, rd_b, sem_b)

            return carry

        lax.fori_loop(0, n_pairs, pair, 0)
        drain(n_chunks - 1, rs_a, rd_a, sem_a)
        compute(n_chunks - 1, rs_a, rd_a)

        pltpu.sync_copy(out_v, out_hbm.at[wid])

    return body(z, src2d, dst2d)


def kernel(z, edge_index):
    n_edges = edge_index.shape[1]
    n_chunks = n_edges // (_NW * _CHUNK)
    ei = edge_index.astype(jnp.int32)
    src3d = ei[0].reshape(_NW, n_chunks, _CHUNK)
    dst3d = ei[1].reshape(_NW, n_chunks, _CHUNK)
    out3d = _decode(z, src3d, dst3d, n_edges=n_edges, n_chunks=n_chunks)
    return out3d.reshape(n_edges)


# speedup vs baseline: 8.5041x; 6.3518x over previous
import functools

import jax
import jax.numpy as jnp
from jax import lax
from jax.experimental import pallas as pl
from jax.experimental.pallas import tpu as pltpu
from jax.experimental.pallas import tpu_sc as plsc

_LANES = 16
_NW = 32
_NC = 2
_CHUNK = 80


@functools.partial(jax.jit, static_argnames=("n_edges", "n_chunks"))
def _decode(z, src2d, dst2d, *, n_edges, n_chunks):
    mesh = plsc.VectorSubcoreMesh(core_axis_name="c", subcore_axis_name="s")

    @functools.partial(
        pl.kernel,
        mesh=mesh,
        compiler_params=pltpu.CompilerParams(needs_layout_passes=False),
        out_type=jax.ShapeDtypeStruct((_NW, n_chunks, _CHUNK), jnp.float32),
        scratch_types=dict(
            src_idx=pltpu.VMEM((n_chunks, _CHUNK), jnp.int32),
            dst_idx=pltpu.VMEM((n_chunks, _CHUNK), jnp.int32),
            rs_a=pltpu.VMEM((_CHUNK, 128), jnp.float32),
            rd_a=pltpu.VMEM((_CHUNK, 128), jnp.float32),
            rs_b=pltpu.VMEM((_CHUNK, 128), jnp.float32),
            rd_b=pltpu.VMEM((_CHUNK, 128), jnp.float32),
            out_v=pltpu.VMEM((n_chunks, _CHUNK), jnp.float32),
            sem_a=pltpu.SemaphoreType.DMA,
            sem_b=pltpu.SemaphoreType.DMA,
        ),
    )
    def body(z_hbm, src_hbm, dst_hbm, out_hbm,
             src_idx, dst_idx, rs_a, rd_a, rs_b, rd_b, out_v, sem_a, sem_b):
        wid = lax.axis_index("s") * _NC + lax.axis_index("c")

        pltpu.sync_copy(src_hbm.at[wid], src_idx)
        pltpu.sync_copy(dst_hbm.at[wid], dst_idx)

        def issue(l, rs, rd, sem):
            pltpu.async_copy(z_hbm.at[src_idx.at[l]], rs, sem)
            pltpu.async_copy(z_hbm.at[dst_idx.at[l]], rd, sem)

        def drain(l, rs, rd, sem):
            pltpu.make_async_copy(z_hbm.at[src_idx.at[l]], rs, sem).wait()
            pltpu.make_async_copy(z_hbm.at[dst_idx.at[l]], rd, sem).wait()

        def compute(l, rs, rd):
            out_row = out_v.at[l]
            zero = jnp.zeros((_LANES,), jnp.float32)
            iota = lax.iota(jnp.int32, _LANES)
            for g in range(_CHUNK // _LANES):
                row = g * _LANES + iota

                def dstep(i, accs):
                    a0, a1, a2, a3 = accs
                    cb = jnp.full((_LANES,), i * 16, jnp.int32) + iota
                    prods = []
                    for k in range(16):
                        col = (cb + k) & 127
                        s = plsc.load_gather(rs, [row, col])
                        t = plsc.load_gather(rd, [row, col])
                        prods.append(s * t)
                    for k in range(0, 16, 4):
                        a0 = a0 + prods[k]
                        a1 = a1 + prods[k + 1]
                        a2 = a2 + prods[k + 2]
                        a3 = a3 + prods[k + 3]
                    return a0, a1, a2, a3

                a0, a1, a2, a3 = lax.fori_loop(0, 128 // 16, dstep,
                                               (zero, zero, zero, zero))
                acc = (a0 + a1) + (a2 + a3)
                out_row[pl.ds(g * _LANES, _LANES)] = 1.0 / (1.0 + jnp.exp(-acc))

        issue(0, rs_a, rd_a, sem_a)
        issue(1, rs_b, rd_b, sem_b)
        n_pairs = (n_chunks - 1) // 2

        def pair(j, carry):
            drain(2 * j, rs_a, rd_a, sem_a)
            compute(2 * j, rs_a, rd_a)
            issue(2 * j + 2, rs_a, rd_a, sem_a)
            drain(2 * j + 1, rs_b, rd_b, sem_b)
            compute(2 * j + 1, rs_b, rd_b)

            @pl.when(j < n_pairs - 1)
            def _():
                issue(2 * j + 3, rs_b, rd_b, sem_b)

            return carry

        lax.fori_loop(0, n_pairs, pair, 0)
        drain(n_chunks - 1, rs_a, rd_a, sem_a)
        compute(n_chunks - 1, rs_a, rd_a)

        pltpu.sync_copy(out_v, out_hbm.at[wid])

    return body(z, src2d, dst2d)


def kernel(z, edge_index):
    n_edges = edge_index.shape[1]
    n_chunks = n_edges // (_NW * _CHUNK)
    ei = edge_index.astype(jnp.int32)
    src3d = ei[0].reshape(_NW, n_chunks, _CHUNK)
    dst3d = ei[1].reshape(_NW, n_chunks, _CHUNK)
    out3d = _decode(z, src3d, dst3d, n_edges=n_edges, n_chunks=n_chunks)
    return out3d.reshape(n_edges)


# speedup vs baseline: 9.7133x; 1.1422x over previous
import functools

import jax
import jax.numpy as jnp
from jax import lax
from jax.experimental import pallas as pl
from jax.experimental.pallas import tpu as pltpu
from jax.experimental.pallas import tpu_sc as plsc

_LANES = 16
_NW = 32
_NC = 2
_CHUNK = 80


@functools.partial(jax.jit, static_argnames=("n_edges", "n_chunks"))
def _decode(z, src2d, dst2d, *, n_edges, n_chunks):
    mesh = plsc.VectorSubcoreMesh(core_axis_name="c", subcore_axis_name="s")

    @functools.partial(
        pl.kernel,
        mesh=mesh,
        compiler_params=pltpu.CompilerParams(needs_layout_passes=False,
                                             use_tc_tiling_on_sc=False),
        out_type=jax.ShapeDtypeStruct((_NW, n_chunks, _CHUNK), jnp.float32),
        scratch_types=dict(
            src_idx=pltpu.VMEM((n_chunks, _CHUNK), jnp.int32),
            dst_idx=pltpu.VMEM((n_chunks, _CHUNK), jnp.int32),
            rs_a=pltpu.VMEM((_CHUNK, 64), jnp.int32),
            rd_a=pltpu.VMEM((_CHUNK, 64), jnp.int32),
            rs_b=pltpu.VMEM((_CHUNK, 64), jnp.int32),
            rd_b=pltpu.VMEM((_CHUNK, 64), jnp.int32),
            out_v=pltpu.VMEM((n_chunks, _CHUNK), jnp.float32),
            sem_a=pltpu.SemaphoreType.DMA,
            sem_b=pltpu.SemaphoreType.DMA,
        ),
    )
    def body(z_hbm, src_hbm, dst_hbm, out_hbm,
             src_idx, dst_idx, rs_a, rd_a, rs_b, rd_b, out_v, sem_a, sem_b):
        wid = lax.axis_index("s") * _NC + lax.axis_index("c")

        pltpu.sync_copy(src_hbm.at[wid], src_idx)
        pltpu.sync_copy(dst_hbm.at[wid], dst_idx)

        def issue(l, rs, rd, sem):
            pltpu.async_copy(z_hbm.at[src_idx.at[l]], rs, sem)
            pltpu.async_copy(z_hbm.at[dst_idx.at[l]], rd, sem)

        def drain(l, rs, rd, sem):
            pltpu.make_async_copy(z_hbm.at[src_idx.at[l]], rs, sem).wait()
            pltpu.make_async_copy(z_hbm.at[dst_idx.at[l]], rd, sem).wait()

        def compute(l, rs, rd):
            out_row = out_v.at[l]
            zero = jnp.zeros((_LANES,), jnp.float32)
            iota = lax.iota(jnp.int32, _LANES)
            for g in range(_CHUNK // _LANES):
                row = g * _LANES + iota

                def dstep(i, accs):
                    a0, a1, a2, a3 = accs
                    cb = jnp.full((_LANES,), i * 16, jnp.int32) + iota
                    prods = []
                    for k in range(16):
                        col = (cb + k) & 63
                        s = plsc.load_gather(rs, [row, col])
                        t = plsc.load_gather(rd, [row, col])
                        p = (plsc.bitcast(s, jnp.bfloat16)
                             * plsc.bitcast(t, jnp.bfloat16))
                        p0, p1 = plsc.unpack(p, format=plsc.PackFormat.INTERLEAVED)
                        prods.append(p0 + p1)
                    for k in range(0, 16, 4):
                        a0 = a0 + prods[k]
                        a1 = a1 + prods[k + 1]
                        a2 = a2 + prods[k + 2]
                        a3 = a3 + prods[k + 3]
                    return a0, a1, a2, a3

                a0, a1, a2, a3 = lax.fori_loop(0, 64 // 16, dstep,
                                               (zero, zero, zero, zero))
                acc = (a0 + a1) + (a2 + a3)
                out_row[pl.ds(g * _LANES, _LANES)] = 1.0 / (1.0 + jnp.exp(-acc))

        issue(0, rs_a, rd_a, sem_a)
        issue(1, rs_b, rd_b, sem_b)
        n_pairs = (n_chunks - 1) // 2

        def pair(j, carry):
            drain(2 * j, rs_a, rd_a, sem_a)
            compute(2 * j, rs_a, rd_a)
            issue(2 * j + 2, rs_a, rd_a, sem_a)
            drain(2 * j + 1, rs_b, rd_b, sem_b)
            compute(2 * j + 1, rs_b, rd_b)

            @pl.when(j < n_pairs - 1)
            def _():
                issue(2 * j + 3, rs_b, rd_b, sem_b)

            return carry

        lax.fori_loop(0, n_pairs, pair, 0)
        drain(n_chunks - 1, rs_a, rd_a, sem_a)
        compute(n_chunks - 1, rs_a, rd_a)

        pltpu.sync_copy(out_v, out_hbm.at[wid])

    return body(z, src2d, dst2d)


def kernel(z, edge_index):
    n_edges = edge_index.shape[1]
    n_chunks = n_edges // (_NW * _CHUNK)
    ei = edge_index.astype(jnp.int32)
    src3d = ei[0].reshape(_NW, n_chunks, _CHUNK)
    dst3d = ei[1].reshape(_NW, n_chunks, _CHUNK)
    n_nodes, d_model = z.shape
    z_pk = jax.lax.bitcast_convert_type(
        z.astype(jnp.bfloat16).reshape(n_nodes, d_model // 2, 2), jnp.int32)
    out3d = _decode(z_pk, src3d, dst3d, n_edges=n_edges, n_chunks=n_chunks)
    return out3d.reshape(n_edges)
